# Initial kernel scaffold; baseline (speedup 1.0000x reference)
import jax
import jax.numpy as jnp
from jax.experimental import pallas as pl


def kernel(tokens, table):
    raise NotImplementedError("write your pallas kernel here")


# speedup vs baseline: 1.4775x; 1.4775x over previous
import functools
import math

import jax
import jax.numpy as jnp
from jax import lax
from jax.experimental import pallas as pl
from jax.experimental.pallas import tpu as pltpu
from jax.experimental.pallas import tpu_sc as plsc

_B = 4096 * 200
_D = 32
_NW = 32
_BPW = _B // _NW
_CH = 128
_G = 10
_R = _CH * _G
_NG = _BPW // _R
_CPW = _BPW // _CH
_SCALE = math.sqrt(float(_D))

_mesh = plsc.VectorSubcoreMesh(core_axis_name="c", subcore_axis_name="s")


def _scale_group(buf):
    rows_per_iter = 8

    def body(i, carry):
        base = i * rows_per_iter
        for k in range(rows_per_iter):
            for h in range(2):
                sl = pl.ds(h * 16, 16)
                buf[base + k, sl] = buf[base + k, sl] * _SCALE
        return carry

    lax.fori_loop(0, _R // rows_per_iter, body, 0)


@functools.partial(
    pl.kernel,
    out_type=jax.ShapeDtypeStruct((_B, _D), jnp.float32),
    mesh=_mesh,
    compiler_params=pltpu.CompilerParams(use_tc_tiling_on_sc=False),
    scratch_types=[
        pltpu.VMEM((_CPW, _CH), jnp.int32),
        pltpu.VMEM((2, _R, _D), jnp.float32),
        pltpu.SemaphoreType.DMA,
        pltpu.SemaphoreType.DMA,
    ],
)
def _emb_lookup(tokens_hbm, table_hbm, out_hbm, idx_v, rows_v, gsem, wsem):
    wid = lax.axis_index("s") * 2 + lax.axis_index("c")
    pltpu.sync_copy(tokens_hbm.at[pl.ds(wid * _CPW, _CPW)], idx_v)

    gathers = {}
    writes = {}

    def fire(g):
        b = g % 2
        descs = []
        for k in range(_G):
            chunk = g * _G + k
            descs.append(
                pltpu.async_copy(
                    table_hbm.at[idx_v.at[chunk]],
                    rows_v.at[b, pl.ds(k * _CH, _CH)],
                    gsem,
                )
            )
        gathers[g] = descs

    fire(0)
    for g in range(_NG):
        b = g % 2
        for d in gathers.pop(g):
            d.wait()
        if g + 1 < _NG:
            if g - 1 >= 0:
                writes.pop(g - 1).wait()
            fire(g + 1)
        _scale_group(rows_v.at[b])
        writes[g] = pltpu.async_copy(
            rows_v.at[b],
            out_hbm.at[pl.ds(wid * _BPW + g * _R, _R)],
            wsem,
        )
    for g in sorted(writes):
        writes[g].wait()


def kernel(tokens, table):
    idx = tokens.reshape(-1).astype(jnp.int32).reshape(_B // _CH, _CH)
    out = _emb_lookup(idx, table)
    return out.reshape(tokens.shape + (_D,))
